# baseline (speedup 1.0000x reference)
import jax
import jax.numpy as jnp
from jax.experimental import pallas as pl
from jax.experimental.pallas import tpu as pltpu

NUM_CORES = 2
IMG_BLOCK = 64
DEC_TK = 6272


def _conv_stack_kernel(x_ref, w1_ref, s1_ref, c1_ref,
                       w2_ref, s2_ref, c2_ref,
                       w3_ref, s3_ref, c3_ref,
                       o_ref, p1_ref, p2_ref, p3_ref):
    B, H, W = o_ref.shape[0], o_ref.shape[1], o_ref.shape[2]
    Hp, Wp = H + 2, W + 2

    def zero_rows(p_ref):
        c = p_ref.shape[-1]
        zrow = jnp.zeros((B, 1, Wp, c), jnp.bfloat16)
        p_ref[0:B, 0:1, :, :] = zrow
        p_ref[0:B, H + 1:H + 2, :, :] = zrow

    def conv_parts(p_ref, w_ref):
        cin = p_ref.shape[-1]
        cout = w_ref.shape[3]
        Mo = B * Hp * Wp
        xm = p_ref[...].reshape((B + 1) * Hp * Wp, cin)
        a3w = jnp.concatenate(
            [xm[0:Mo], xm[Wp:Wp + Mo], xm[2 * Wp:2 * Wp + Mo]], axis=1)
        parts = []
        for dx in range(3):
            wcat = w_ref[:, dx].reshape(3 * cin, cout).astype(jnp.bfloat16)
            part = jnp.dot(a3w, wcat, preferred_element_type=jnp.float32)
            parts.append(part.reshape(B, Hp, Wp, cout)[:, 0:H, :, :])
        return parts

    def bn_relu(v, s_ref, c_ref):
        cout = s_ref.shape[-1]
        s = s_ref[...].reshape(1, 1, 1, cout).astype(v.dtype)
        c = c_ref[...].reshape(1, 1, 1, cout).astype(v.dtype)
        return jnp.maximum(v * s + c, 0.0).astype(jnp.bfloat16)

    def layer_to_pad(p_src, w_ref, s_ref, c_ref, p_dst):
        parts = conv_parts(p_src, w_ref)
        val = (jnp.roll(parts[0], 1, axis=2) + parts[1]
               + jnp.roll(parts[2], -1, axis=2))
        y = bn_relu(val, s_ref, c_ref)
        p_dst[0:B, 1:H + 1, :, :] = y
        cout = y.shape[-1]
        zcol = jnp.zeros((B, H, 1, cout), jnp.bfloat16)
        p_dst[0:B, 1:H + 1, 0:1, :] = zcol
        p_dst[0:B, 1:H + 1, W + 1:W + 2, :] = zcol

    zero_rows(p2_ref)
    zero_rows(p3_ref)
    zero_rows(p1_ref)
    zcol1 = jnp.zeros((B, H, 1, p1_ref.shape[-1]), jnp.bfloat16)
    p1_ref[0:B, 1:H + 1, 0:1, :] = zcol1
    p1_ref[0:B, 1:H + 1, W + 1:W + 2, :] = zcol1
    p1_ref[0:B, 1:H + 1, 1:W + 1, :] = x_ref[...].astype(jnp.bfloat16)

    layer_to_pad(p1_ref, w1_ref, s1_ref, c1_ref, p2_ref)
    layer_to_pad(p2_ref, w2_ref, s2_ref, c2_ref, p3_ref)

    parts = conv_parts(p3_ref, w3_ref)
    acc = (parts[0][:, :, 0:W, :] + parts[1][:, :, 1:W + 1, :]
           + parts[2][:, :, 2:W + 2, :])
    o_ref[...] = bn_relu(acc, s3_ref, c3_ref)


def _conv_stack(x_nhwc, w1, s1, c1, w2, s2, c2, w3, s3, c3):
    N, H, W, Cin = x_nhwc.shape
    Hp, Wp = H + 2, W + 2
    B = IMG_BLOCK
    JPC = N // B // NUM_CORES
    return pl.pallas_call(
        _conv_stack_kernel,
        out_shape=jax.ShapeDtypeStruct((N, H, W, 128), jnp.bfloat16),
        grid=(NUM_CORES, N // B // NUM_CORES),
        in_specs=[
            pl.BlockSpec((B, H, W, Cin), lambda c, j: (c * JPC + j, 0, 0, 0)),
            pl.BlockSpec((3, 3, Cin, 32), lambda c, j: (0, 0, 0, 0)),
            pl.BlockSpec((1, 32), lambda c, j: (0, 0)),
            pl.BlockSpec((1, 32), lambda c, j: (0, 0)),
            pl.BlockSpec((3, 3, 32, 64), lambda c, j: (0, 0, 0, 0)),
            pl.BlockSpec((1, 64), lambda c, j: (0, 0)),
            pl.BlockSpec((1, 64), lambda c, j: (0, 0)),
            pl.BlockSpec((3, 3, 64, 128), lambda c, j: (0, 0, 0, 0)),
            pl.BlockSpec((1, 128), lambda c, j: (0, 0)),
            pl.BlockSpec((1, 128), lambda c, j: (0, 0)),
        ],
        out_specs=pl.BlockSpec((B, H, W, 128),
                               lambda c, j: (c * JPC + j, 0, 0, 0)),
        scratch_shapes=[
            pltpu.VMEM((B + 1, Hp, Wp, Cin), jnp.bfloat16),
            pltpu.VMEM((B + 1, Hp, Wp, 32), jnp.bfloat16),
            pltpu.VMEM((B + 1, Hp, Wp, 64), jnp.bfloat16),
        ],
        compiler_params=pltpu.CompilerParams(
            dimension_semantics=("parallel", "arbitrary")),
    )(x_nhwc, w1, s1, c1, w2, s2, c2, w3, s3, c3)


def _decoder_kernel(x_ref, w1_ref, b1_ref, w2_ref, o_ref, acc_ref):
    k = pl.program_id(1)

    @pl.when(k == 0)
    def _():
        acc_ref[...] = jnp.zeros_like(acc_ref)

    acc_ref[...] += jnp.dot(x_ref[...], w1_ref[0],
                            preferred_element_type=jnp.float32)

    @pl.when(k == pl.num_programs(1) - 1)
    def _():
        h = jax.nn.sigmoid(acc_ref[...] + b1_ref[...])
        o_ref[0] = jnp.dot(h, w2_ref[...],
                           preferred_element_type=jnp.float32)


def _decoder(x, dw1, db1, dw2, db2):
    B, K = x.shape
    n_half, Kw, hh = dw1.shape
    C = dw2.shape[1]
    tk = DEC_TK
    partial = pl.pallas_call(
        _decoder_kernel,
        out_shape=jax.ShapeDtypeStruct((n_half, B, C), jnp.float32),
        grid=(n_half, K // tk),
        in_specs=[
            pl.BlockSpec((B, tk), lambda h, k: (0, k)),
            pl.BlockSpec((1, tk, hh), lambda h, k: (h, k, 0)),
            pl.BlockSpec((1, hh), lambda h, k: (0, h)),
            pl.BlockSpec((hh, C), lambda h, k: (h, 0)),
        ],
        out_specs=pl.BlockSpec((1, B, C), lambda h, k: (h, 0, 0)),
        scratch_shapes=[pltpu.VMEM((B, hh), jnp.float32)],
        compiler_params=pltpu.CompilerParams(
            dimension_semantics=("parallel", "arbitrary"),
            vmem_limit_bytes=48 << 20),
    )(x, dw1, db1, dw2)
    return jnp.sum(partial, axis=0) + db2


@jax.jit
def kernel(x_nchw, w1, s1, c1, w2, s2, c2, w3, s3, c3, dw1, db1, dw2, db2):
    x = jnp.transpose(x_nchw, (0, 2, 3, 1))
    x = _conv_stack(x, w1, s1, c1, w2, s2, c2, w3, s3, c3)
    x = x.reshape(x.shape[0], -1)
    return _decoder(x, dw1, db1, dw2, db2)


# speedup vs baseline: 1.0137x; 1.0137x over previous
import jax
import jax.numpy as jnp
from jax.experimental import pallas as pl
from jax.experimental.pallas import tpu as pltpu

NUM_CORES = 2
IMG_BLOCK = 64
DEC_TK = 6272


def _conv_stack_kernel(x_ref, w1_ref, s1_ref, c1_ref,
                       w2_ref, s2_ref, c2_ref,
                       w3_ref, s3_ref, c3_ref,
                       o_ref, p1_ref, p2_ref, p3_ref):
    B, H, W = o_ref.shape[0], o_ref.shape[1], o_ref.shape[2]
    Hp, Wp = H + 2, W + 2

    def zero_rows(p_ref):
        c = p_ref.shape[-1]
        zrow = jnp.zeros((B, 1, Wp, c), jnp.bfloat16)
        p_ref[0:B, 0:1, :, :] = zrow
        p_ref[0:B, H + 1:H + 2, :, :] = zrow

    def conv_parts(p_ref, w_ref):
        cin = p_ref.shape[-1]
        cout = w_ref.shape[3]
        Mo = B * Hp * Wp
        xm = p_ref[...].reshape((B + 1) * Hp * Wp, cin)
        a3w = jnp.concatenate(
            [xm[0:Mo], xm[Wp:Wp + Mo], xm[2 * Wp:2 * Wp + Mo]], axis=1)
        parts = []
        for dx in range(3):
            wcat = w_ref[:, dx].reshape(3 * cin, cout).astype(jnp.bfloat16)
            part = jnp.dot(a3w, wcat, preferred_element_type=jnp.float32)
            parts.append(part.reshape(B, Hp, Wp, cout)[:, 0:H, :, :])
        return parts

    def bn_relu(v, s_ref, c_ref):
        cout = s_ref.shape[-1]
        s = s_ref[...].reshape(1, 1, 1, cout).astype(v.dtype)
        c = c_ref[...].reshape(1, 1, 1, cout).astype(v.dtype)
        return jnp.maximum(v * s + c, 0.0).astype(jnp.bfloat16)

    def layer_to_pad(p_src, w_ref, s_ref, c_ref, p_dst):
        parts = conv_parts(p_src, w_ref)
        val = (jnp.roll(parts[0], 1, axis=2) + parts[1]
               + jnp.roll(parts[2], -1, axis=2))
        y = bn_relu(val, s_ref, c_ref)
        p_dst[0:B, 1:H + 1, :, :] = y
        cout = y.shape[-1]
        zcol = jnp.zeros((B, H, 1, cout), jnp.bfloat16)
        p_dst[0:B, 1:H + 1, 0:1, :] = zcol
        p_dst[0:B, 1:H + 1, W + 1:W + 2, :] = zcol

    zero_rows(p2_ref)
    zero_rows(p3_ref)
    zero_rows(p1_ref)
    zcol1 = jnp.zeros((B, H, 1, p1_ref.shape[-1]), jnp.bfloat16)
    p1_ref[0:B, 1:H + 1, 0:1, :] = zcol1
    p1_ref[0:B, 1:H + 1, W + 1:W + 2, :] = zcol1
    p1_ref[0:B, 1:H + 1, 1:W + 1, :] = x_ref[...].astype(jnp.bfloat16)

    layer_to_pad(p1_ref, w1_ref, s1_ref, c1_ref, p2_ref)
    layer_to_pad(p2_ref, w2_ref, s2_ref, c2_ref, p3_ref)

    parts = conv_parts(p3_ref, w3_ref)
    acc = (parts[0][:, :, 0:W, :] + parts[1][:, :, 1:W + 1, :]
           + parts[2][:, :, 2:W + 2, :])
    o_ref[...] = bn_relu(acc, s3_ref, c3_ref)


def _conv_stack(x_nhwc, w1, s1, c1, w2, s2, c2, w3, s3, c3):
    N, H, W, Cin = x_nhwc.shape
    Hp, Wp = H + 2, W + 2
    B = IMG_BLOCK
    JPC = N // B // NUM_CORES
    return pl.pallas_call(
        _conv_stack_kernel,
        out_shape=jax.ShapeDtypeStruct((N, H, W, 128), jnp.bfloat16),
        grid=(NUM_CORES, N // B // NUM_CORES),
        in_specs=[
            pl.BlockSpec((B, H, W, Cin), lambda c, j: (c * JPC + j, 0, 0, 0)),
            pl.BlockSpec((3, 3, Cin, 32), lambda c, j: (0, 0, 0, 0)),
            pl.BlockSpec((1, 32), lambda c, j: (0, 0)),
            pl.BlockSpec((1, 32), lambda c, j: (0, 0)),
            pl.BlockSpec((3, 3, 32, 64), lambda c, j: (0, 0, 0, 0)),
            pl.BlockSpec((1, 64), lambda c, j: (0, 0)),
            pl.BlockSpec((1, 64), lambda c, j: (0, 0)),
            pl.BlockSpec((3, 3, 64, 128), lambda c, j: (0, 0, 0, 0)),
            pl.BlockSpec((1, 128), lambda c, j: (0, 0)),
            pl.BlockSpec((1, 128), lambda c, j: (0, 0)),
        ],
        out_specs=pl.BlockSpec((B, H, W, 128),
                               lambda c, j: (c * JPC + j, 0, 0, 0)),
        scratch_shapes=[
            pltpu.VMEM((B + 1, Hp, Wp, Cin), jnp.bfloat16),
            pltpu.VMEM((B + 1, Hp, Wp, 32), jnp.bfloat16),
            pltpu.VMEM((B + 1, Hp, Wp, 64), jnp.bfloat16),
        ],
        compiler_params=pltpu.CompilerParams(
            dimension_semantics=("parallel", "arbitrary")),
    )(x_nhwc, w1, s1, c1, w2, s2, c2, w3, s3, c3)


def _decoder_kernel(x_ref, w1_ref, b1_ref, w2_ref, b2_ref, o_ref, acc_ref):
    k = pl.program_id(0)
    hh = w1_ref.shape[2]

    @pl.when(k == 0)
    def _():
        acc_ref[...] = jnp.zeros_like(acc_ref)

    acc_ref[:, 0:hh] += jnp.dot(x_ref[...], w1_ref[0],
                                preferred_element_type=jnp.float32)
    acc_ref[:, hh:2 * hh] += jnp.dot(x_ref[...], w1_ref[1],
                                     preferred_element_type=jnp.float32)

    @pl.when(k == pl.num_programs(0) - 1)
    def _():
        h = jax.nn.sigmoid(acc_ref[...] + b1_ref[...])
        o_ref[...] = jnp.dot(h, w2_ref[...],
                             preferred_element_type=jnp.float32) + b2_ref[...]


def _decoder(x, dw1, db1, dw2, db2):
    B, K = x.shape
    n_half, Kw, hh = dw1.shape
    Hd = n_half * hh
    C = dw2.shape[1]
    tk = DEC_TK
    return pl.pallas_call(
        _decoder_kernel,
        out_shape=jax.ShapeDtypeStruct((B, C), jnp.float32),
        grid=(K // tk,),
        in_specs=[
            pl.BlockSpec((B, tk), lambda k: (0, k)),
            pl.BlockSpec((n_half, tk, hh), lambda k: (0, k, 0)),
            pl.BlockSpec((1, Hd), lambda k: (0, 0)),
            pl.BlockSpec((Hd, C), lambda k: (0, 0)),
            pl.BlockSpec((1, C), lambda k: (0, 0)),
        ],
        out_specs=pl.BlockSpec((B, C), lambda k: (0, 0)),
        scratch_shapes=[pltpu.VMEM((B, Hd), jnp.float32)],
        compiler_params=pltpu.CompilerParams(
            dimension_semantics=("arbitrary",),
            vmem_limit_bytes=48 << 20),
    )(x, dw1, db1, dw2, db2)


@jax.jit
def kernel(x_nchw, w1, s1, c1, w2, s2, c2, w3, s3, c3, dw1, db1, dw2, db2):
    x = jnp.transpose(x_nchw, (0, 2, 3, 1))
    x = _conv_stack(x, w1, s1, c1, w2, s2, c2, w3, s3, c3)
    x = x.reshape(x.shape[0], -1)
    return _decoder(x, dw1, db1, dw2, db2)


# speedup vs baseline: 1.0164x; 1.0027x over previous
import jax
import jax.numpy as jnp
from jax.experimental import pallas as pl
from jax.experimental.pallas import tpu as pltpu

NUM_CORES = 2
IMG_BLOCK = 64
DEC_TK = 3584


def _conv_stack_kernel(x_ref, w1_ref, s1_ref, c1_ref,
                       w2_ref, s2_ref, c2_ref,
                       w3_ref, s3_ref, c3_ref,
                       o_ref, p1_ref, p2_ref, p3_ref):
    B, H, W = o_ref.shape[0], o_ref.shape[1], o_ref.shape[2]
    Hp, Wp = H + 2, W + 2

    def zero_rows(p_ref):
        c = p_ref.shape[-1]
        zrow = jnp.zeros((B, 1, Wp, c), jnp.bfloat16)
        p_ref[0:B, 0:1, :, :] = zrow
        p_ref[0:B, H + 1:H + 2, :, :] = zrow

    def conv_parts(p_ref, w_ref):
        cin = p_ref.shape[-1]
        cout = w_ref.shape[3]
        Mo = B * Hp * Wp
        xm = p_ref[...].reshape((B + 1) * Hp * Wp, cin)
        a3w = jnp.concatenate(
            [xm[0:Mo], xm[Wp:Wp + Mo], xm[2 * Wp:2 * Wp + Mo]], axis=1)
        parts = []
        for dx in range(3):
            wcat = w_ref[:, dx].reshape(3 * cin, cout).astype(jnp.bfloat16)
            part = jnp.dot(a3w, wcat, preferred_element_type=jnp.float32)
            parts.append(part.reshape(B, Hp, Wp, cout)[:, 0:H, :, :])
        return parts

    def bn_relu(v, s_ref, c_ref):
        cout = s_ref.shape[-1]
        s = s_ref[...].reshape(1, 1, 1, cout).astype(v.dtype)
        c = c_ref[...].reshape(1, 1, 1, cout).astype(v.dtype)
        return jnp.maximum(v * s + c, 0.0).astype(jnp.bfloat16)

    def layer_to_pad(p_src, w_ref, s_ref, c_ref, p_dst):
        parts = conv_parts(p_src, w_ref)
        val = (jnp.roll(parts[0], 1, axis=2) + parts[1]
               + jnp.roll(parts[2], -1, axis=2))
        y = bn_relu(val, s_ref, c_ref)
        p_dst[0:B, 1:H + 1, :, :] = y
        cout = y.shape[-1]
        zcol = jnp.zeros((B, H, 1, cout), jnp.bfloat16)
        p_dst[0:B, 1:H + 1, 0:1, :] = zcol
        p_dst[0:B, 1:H + 1, W + 1:W + 2, :] = zcol

    zero_rows(p2_ref)
    zero_rows(p3_ref)
    zero_rows(p1_ref)
    zcol1 = jnp.zeros((B, H, 1, p1_ref.shape[-1]), jnp.bfloat16)
    p1_ref[0:B, 1:H + 1, 0:1, :] = zcol1
    p1_ref[0:B, 1:H + 1, W + 1:W + 2, :] = zcol1
    p1_ref[0:B, 1:H + 1, 1:W + 1, :] = x_ref[...].astype(jnp.bfloat16)

    layer_to_pad(p1_ref, w1_ref, s1_ref, c1_ref, p2_ref)
    layer_to_pad(p2_ref, w2_ref, s2_ref, c2_ref, p3_ref)

    parts = conv_parts(p3_ref, w3_ref)
    acc = (parts[0][:, :, 0:W, :] + parts[1][:, :, 1:W + 1, :]
           + parts[2][:, :, 2:W + 2, :])
    o_ref[...] = bn_relu(acc, s3_ref, c3_ref)


def _conv_stack(x_nhwc, w1, s1, c1, w2, s2, c2, w3, s3, c3):
    N, H, W, Cin = x_nhwc.shape
    Hp, Wp = H + 2, W + 2
    B = IMG_BLOCK
    JPC = N // B // NUM_CORES
    return pl.pallas_call(
        _conv_stack_kernel,
        out_shape=jax.ShapeDtypeStruct((N, H, W, 128), jnp.bfloat16),
        grid=(NUM_CORES, N // B // NUM_CORES),
        in_specs=[
            pl.BlockSpec((B, H, W, Cin), lambda c, j: (c * JPC + j, 0, 0, 0)),
            pl.BlockSpec((3, 3, Cin, 32), lambda c, j: (0, 0, 0, 0)),
            pl.BlockSpec((1, 32), lambda c, j: (0, 0)),
            pl.BlockSpec((1, 32), lambda c, j: (0, 0)),
            pl.BlockSpec((3, 3, 32, 64), lambda c, j: (0, 0, 0, 0)),
            pl.BlockSpec((1, 64), lambda c, j: (0, 0)),
            pl.BlockSpec((1, 64), lambda c, j: (0, 0)),
            pl.BlockSpec((3, 3, 64, 128), lambda c, j: (0, 0, 0, 0)),
            pl.BlockSpec((1, 128), lambda c, j: (0, 0)),
            pl.BlockSpec((1, 128), lambda c, j: (0, 0)),
        ],
        out_specs=pl.BlockSpec((B, H, W, 128),
                               lambda c, j: (c * JPC + j, 0, 0, 0)),
        scratch_shapes=[
            pltpu.VMEM((B + 1, Hp, Wp, Cin), jnp.bfloat16),
            pltpu.VMEM((B + 1, Hp, Wp, 32), jnp.bfloat16),
            pltpu.VMEM((B + 1, Hp, Wp, 64), jnp.bfloat16),
        ],
        compiler_params=pltpu.CompilerParams(
            dimension_semantics=("parallel", "arbitrary")),
    )(x_nhwc, w1, s1, c1, w2, s2, c2, w3, s3, c3)


def _decoder_kernel(x_ref, w1_ref, b1_ref, w2_ref, b2_ref, o_ref, acc_ref):
    k = pl.program_id(0)
    hh = w1_ref.shape[2]

    @pl.when(k == 0)
    def _():
        acc_ref[...] = jnp.zeros_like(acc_ref)

    acc_ref[:, 0:hh] += jnp.dot(x_ref[...], w1_ref[0],
                                preferred_element_type=jnp.float32)
    acc_ref[:, hh:2 * hh] += jnp.dot(x_ref[...], w1_ref[1],
                                     preferred_element_type=jnp.float32)

    @pl.when(k == pl.num_programs(0) - 1)
    def _():
        h = jax.nn.sigmoid(acc_ref[...] + b1_ref[...])
        o_ref[...] = jnp.dot(h, w2_ref[...],
                             preferred_element_type=jnp.float32) + b2_ref[...]


def _decoder(x, dw1, db1, dw2, db2):
    B, K = x.shape
    n_half, Kw, hh = dw1.shape
    Hd = n_half * hh
    C = dw2.shape[1]
    tk = DEC_TK
    return pl.pallas_call(
        _decoder_kernel,
        out_shape=jax.ShapeDtypeStruct((B, C), jnp.float32),
        grid=(K // tk,),
        in_specs=[
            pl.BlockSpec((B, tk), lambda k: (0, k)),
            pl.BlockSpec((n_half, tk, hh), lambda k: (0, k, 0)),
            pl.BlockSpec((1, Hd), lambda k: (0, 0)),
            pl.BlockSpec((Hd, C), lambda k: (0, 0)),
            pl.BlockSpec((1, C), lambda k: (0, 0)),
        ],
        out_specs=pl.BlockSpec((B, C), lambda k: (0, 0)),
        scratch_shapes=[pltpu.VMEM((B, Hd), jnp.float32)],
        compiler_params=pltpu.CompilerParams(
            dimension_semantics=("arbitrary",),
            vmem_limit_bytes=48 << 20),
    )(x, dw1, db1, dw2, db2)


@jax.jit
def kernel(x_nchw, w1, s1, c1, w2, s2, c2, w3, s3, c3, dw1, db1, dw2, db2):
    x = jnp.transpose(x_nchw, (0, 2, 3, 1))
    x = _conv_stack(x, w1, s1, c1, w2, s2, c2, w3, s3, c3)
    x = x.reshape(x.shape[0], -1)
    return _decoder(x, dw1, db1, dw2, db2)


# speedup vs baseline: 1.0178x; 1.0014x over previous
import jax
import jax.numpy as jnp
from jax.experimental import pallas as pl
from jax.experimental.pallas import tpu as pltpu

NUM_CORES = 2
IMG_BLOCK = 32
DEC_TK = 3584


def _conv_stack_kernel(x_ref, w1_ref, s1_ref, c1_ref,
                       w2_ref, s2_ref, c2_ref,
                       w3_ref, s3_ref, c3_ref,
                       o_ref, p1_ref, p2_ref, p3_ref):
    B, H, W = o_ref.shape[0], o_ref.shape[1], o_ref.shape[2]
    Hp, Wp = H + 2, W + 2

    def zero_rows(p_ref):
        c = p_ref.shape[-1]
        zrow = jnp.zeros((B, 1, Wp, c), jnp.bfloat16)
        p_ref[0:B, 0:1, :, :] = zrow
        p_ref[0:B, H + 1:H + 2, :, :] = zrow

    def conv_parts(p_ref, w_ref):
        cin = p_ref.shape[-1]
        cout = w_ref.shape[3]
        Mo = B * Hp * Wp
        xm = p_ref[...].reshape((B + 1) * Hp * Wp, cin)
        a3w = jnp.concatenate(
            [xm[0:Mo], xm[Wp:Wp + Mo], xm[2 * Wp:2 * Wp + Mo]], axis=1)
        parts = []
        for dx in range(3):
            wcat = w_ref[:, dx].reshape(3 * cin, cout).astype(jnp.bfloat16)
            part = jnp.dot(a3w, wcat, preferred_element_type=jnp.float32)
            parts.append(part.reshape(B, Hp, Wp, cout)[:, 0:H, :, :])
        return parts

    def bn_relu(v, s_ref, c_ref):
        cout = s_ref.shape[-1]
        s = s_ref[...].reshape(1, 1, 1, cout).astype(v.dtype)
        c = c_ref[...].reshape(1, 1, 1, cout).astype(v.dtype)
        return jnp.maximum(v * s + c, 0.0).astype(jnp.bfloat16)

    def layer_to_pad(p_src, w_ref, s_ref, c_ref, p_dst):
        parts = conv_parts(p_src, w_ref)
        val = (jnp.roll(parts[0], 1, axis=2) + parts[1]
               + jnp.roll(parts[2], -1, axis=2))
        y = bn_relu(val, s_ref, c_ref)
        p_dst[0:B, 1:H + 1, :, :] = y
        cout = y.shape[-1]
        zcol = jnp.zeros((B, H, 1, cout), jnp.bfloat16)
        p_dst[0:B, 1:H + 1, 0:1, :] = zcol
        p_dst[0:B, 1:H + 1, W + 1:W + 2, :] = zcol

    zero_rows(p2_ref)
    zero_rows(p3_ref)
    zero_rows(p1_ref)
    zcol1 = jnp.zeros((B, H, 1, p1_ref.shape[-1]), jnp.bfloat16)
    p1_ref[0:B, 1:H + 1, 0:1, :] = zcol1
    p1_ref[0:B, 1:H + 1, W + 1:W + 2, :] = zcol1
    p1_ref[0:B, 1:H + 1, 1:W + 1, :] = x_ref[...].astype(jnp.bfloat16)

    layer_to_pad(p1_ref, w1_ref, s1_ref, c1_ref, p2_ref)
    layer_to_pad(p2_ref, w2_ref, s2_ref, c2_ref, p3_ref)

    parts = conv_parts(p3_ref, w3_ref)
    acc = (parts[0][:, :, 0:W, :] + parts[1][:, :, 1:W + 1, :]
           + parts[2][:, :, 2:W + 2, :])
    o_ref[...] = bn_relu(acc, s3_ref, c3_ref)


def _conv_stack(x_nhwc, w1, s1, c1, w2, s2, c2, w3, s3, c3):
    N, H, W, Cin = x_nhwc.shape
    Hp, Wp = H + 2, W + 2
    B = IMG_BLOCK
    JPC = N // B // NUM_CORES
    return pl.pallas_call(
        _conv_stack_kernel,
        out_shape=jax.ShapeDtypeStruct((N, H, W, 128), jnp.bfloat16),
        grid=(NUM_CORES, N // B // NUM_CORES),
        in_specs=[
            pl.BlockSpec((B, H, W, Cin), lambda c, j: (c * JPC + j, 0, 0, 0)),
            pl.BlockSpec((3, 3, Cin, 32), lambda c, j: (0, 0, 0, 0)),
            pl.BlockSpec((1, 32), lambda c, j: (0, 0)),
            pl.BlockSpec((1, 32), lambda c, j: (0, 0)),
            pl.BlockSpec((3, 3, 32, 64), lambda c, j: (0, 0, 0, 0)),
            pl.BlockSpec((1, 64), lambda c, j: (0, 0)),
            pl.BlockSpec((1, 64), lambda c, j: (0, 0)),
            pl.BlockSpec((3, 3, 64, 128), lambda c, j: (0, 0, 0, 0)),
            pl.BlockSpec((1, 128), lambda c, j: (0, 0)),
            pl.BlockSpec((1, 128), lambda c, j: (0, 0)),
        ],
        out_specs=pl.BlockSpec((B, H, W, 128),
                               lambda c, j: (c * JPC + j, 0, 0, 0)),
        scratch_shapes=[
            pltpu.VMEM((B + 1, Hp, Wp, Cin), jnp.bfloat16),
            pltpu.VMEM((B + 1, Hp, Wp, 32), jnp.bfloat16),
            pltpu.VMEM((B + 1, Hp, Wp, 64), jnp.bfloat16),
        ],
        compiler_params=pltpu.CompilerParams(
            dimension_semantics=("parallel", "arbitrary")),
    )(x_nhwc, w1, s1, c1, w2, s2, c2, w3, s3, c3)


def _decoder_kernel(x_ref, w1_ref, b1_ref, w2_ref, b2_ref, o_ref, acc_ref):
    k = pl.program_id(0)
    hh = w1_ref.shape[2]

    @pl.when(k == 0)
    def _():
        acc_ref[...] = jnp.zeros_like(acc_ref)

    acc_ref[:, 0:hh] += jnp.dot(x_ref[...], w1_ref[0],
                                preferred_element_type=jnp.float32)
    acc_ref[:, hh:2 * hh] += jnp.dot(x_ref[...], w1_ref[1],
                                     preferred_element_type=jnp.float32)

    @pl.when(k == pl.num_programs(0) - 1)
    def _():
        h = jax.nn.sigmoid(acc_ref[...] + b1_ref[...])
        o_ref[...] = jnp.dot(h, w2_ref[...],
                             preferred_element_type=jnp.float32) + b2_ref[...]


def _decoder(x, dw1, db1, dw2, db2):
    B, K = x.shape
    n_half, Kw, hh = dw1.shape
    Hd = n_half * hh
    C = dw2.shape[1]
    tk = DEC_TK
    return pl.pallas_call(
        _decoder_kernel,
        out_shape=jax.ShapeDtypeStruct((B, C), jnp.float32),
        grid=(K // tk,),
        in_specs=[
            pl.BlockSpec((B, tk), lambda k: (0, k)),
            pl.BlockSpec((n_half, tk, hh), lambda k: (0, k, 0)),
            pl.BlockSpec((1, Hd), lambda k: (0, 0)),
            pl.BlockSpec((Hd, C), lambda k: (0, 0)),
            pl.BlockSpec((1, C), lambda k: (0, 0)),
        ],
        out_specs=pl.BlockSpec((B, C), lambda k: (0, 0)),
        scratch_shapes=[pltpu.VMEM((B, Hd), jnp.float32)],
        compiler_params=pltpu.CompilerParams(
            dimension_semantics=("arbitrary",),
            vmem_limit_bytes=48 << 20),
    )(x, dw1, db1, dw2, db2)


@jax.jit
def kernel(x_nchw, w1, s1, c1, w2, s2, c2, w3, s3, c3, dw1, db1, dw2, db2):
    x = jnp.transpose(x_nchw, (0, 2, 3, 1))
    x = _conv_stack(x, w1, s1, c1, w2, s2, c2, w3, s3, c3)
    x = x.reshape(x.shape[0], -1)
    return _decoder(x, dw1, db1, dw2, db2)
